# baseline (speedup 1.0000x reference)
import jax
import jax.numpy as jnp
from jax import lax
from jax.experimental import pallas as pl
from jax.experimental.pallas import tpu as pltpu
from jax.experimental.pallas import tpu_sc as plsc

_N = 10000
_E = 320000
_D = 128

_NC = 2
_NS = 16
_NW = _NC * _NS
_L = 16

_NPW = 320
_LAST = _N - _NPW * (_NW - 1)
_CH = 1600
_U = 10
_G = 128
_RING = 2048
_ROWS_PER_TILE = _NPW + 1
_HD = _D // 4
_CAP = _E + _G

_params = pltpu.CompilerParams(needs_layout_passes=False)


def _mesh():
    return plsc.VectorSubcoreMesh(core_axis_name="c", subcore_axis_name="s",
                                  num_cores=_NC, num_subcores=_NS)


def _sc_prepass_body(src_hbm, dst_hbm, csrc_out, cdst_out, counts_out,
                     csrc, cdst, src_v, dst_v, cnt_v):
    wid = lax.axis_index("s") * _NC + lax.axis_index("c")
    lo = pl.multiple_of(wid * _NPW, 8)
    base = pl.multiple_of(wid * _CAP, 8)

    pad_src = jnp.zeros((_L,), jnp.int32) + lo
    pad_dst = jnp.full((_L,), _NPW, jnp.int32)

    def ring_init(r, c):
        sl = pl.ds(pl.multiple_of(r * _L, _L), _L)
        csrc[sl] = pad_src
        cdst[sl] = pad_dst
        return c
    lax.fori_loop(0, _RING // _L, ring_init, 0)

    lanes = lax.iota(jnp.int32, _L)

    def flush(fired):
        goff = pl.multiple_of(fired & (_RING - 1), _G)
        dst_off = pl.multiple_of(base + fired, 8)
        pltpu.sync_copy(csrc.at[pl.ds(goff, _G)],
                        csrc_out.at[pl.ds(dst_off, _G)])
        pltpu.sync_copy(cdst.at[pl.ds(goff, _G)],
                        cdst_out.at[pl.ds(dst_off, _G)])

    def chunk_body(c, carry):
        pltpu.sync_copy(src_hbm.at[pl.ds(c * _CH, _CH)], src_v)
        pltpu.sync_copy(dst_hbm.at[pl.ds(c * _CH, _CH)], dst_v)

        def scan_body(i, carry2):
            cnt, fired = carry2
            lov = jnp.full((_L,), lo, jnp.int32)
            ms, mis, dls, ss, csums = [], [], [], [], []
            for u in range(_U):
                off = pl.multiple_of((i * _U + u) * _L, _L)
                d = dst_v[pl.ds(off, _L)]
                s = src_v[pl.ds(off, _L)]
                dl = d - lov
                m = (dl >= 0) & (dl < _NPW)
                mi = jnp.where(m, jnp.ones((_L,), jnp.int32),
                               jnp.zeros((_L,), jnp.int32))
                ms.append(m)
                mis.append(mi)
                dls.append(dl)
                ss.append(s)
                csums.append(plsc.cumsum(mi))
            new_cnt = cnt
            for u in range(_U):
                pos = jnp.where(
                    ms[u],
                    (jnp.full((_L,), new_cnt, jnp.int32) + csums[u] - mis[u])
                    & (_RING - 1),
                    jnp.full((_L,), _RING, jnp.int32) + lanes)
                plsc.store_scatter(csrc, [pos], ss[u])
                plsc.store_scatter(cdst, [pos], dls[u])
                new_cnt = new_cnt + csums[u][_L - 1]

            can_flush = new_cnt - fired >= _G

            @pl.when(can_flush)
            def _():
                flush(fired)

            fired = jnp.where(can_flush, fired + _G, fired)
            return new_cnt, fired

        return lax.fori_loop(0, _CH // (_L * _U), scan_body, carry)

    cnt, fired = lax.fori_loop(
        0, _E // _CH, chunk_body, (jnp.int32(0), jnp.int32(0)))

    for _p in range(3):
        do = fired < cnt

        @pl.when(do)
        def _():
            flush(fired)

        fired = jnp.where(do, fired + _G, fired)

    cnt_v[pl.ds(0, _L)] = jnp.full((_L,), fired, jnp.int32)
    pltpu.sync_copy(cnt_v,
                    counts_out.at[pl.ds(pl.multiple_of(wid * _L, 8), _L)])


def _sc_prepass(src, dst):
    f = pl.kernel(
        _sc_prepass_body,
        out_type=(
            jax.ShapeDtypeStruct((_NW * _CAP,), jnp.int32),
            jax.ShapeDtypeStruct((_NW * _CAP,), jnp.int32),
            jax.ShapeDtypeStruct((_NW * _L,), jnp.int32),
        ),
        mesh=_mesh(),
        scratch_types=[
            pltpu.VMEM((_RING + _L,), jnp.int32),
            pltpu.VMEM((_RING + _L,), jnp.int32),
            pltpu.VMEM((_CH,), jnp.int32),
            pltpu.VMEM((_CH,), jnp.int32),
            pltpu.VMEM((_L,), jnp.int32),
        ],
        compiler_params=_params,
    )
    return f(src, dst)


def _sc_segmax_body(x_hbm, csrc_hbm, cdst_hbm, counts_hbm, out_hbm,
                    agga, aggb, aggc, aggd,
                    idx0, idx1, dst0, dst1, rows0, rows1, cnt_v,
                    semi0, semi1, semg0, semg1):
    wid = lax.axis_index("s") * _NC + lax.axis_index("c")
    lo = pl.multiple_of(wid * _NPW, 8)
    base = pl.multiple_of(wid * _CAP, 8)

    neg_inf = jnp.full((_L,), -jnp.inf, jnp.float32)

    def init_body(i, c):
        sl = pl.ds(pl.multiple_of(i * _L, _L), _L)
        agga[sl] = neg_inf
        aggb[sl] = neg_inf
        aggc[sl] = neg_inf
        aggd[sl] = neg_inf
        return c
    lax.fori_loop(0, _ROWS_PER_TILE * _HD // _L, init_body, 0)

    pltpu.sync_copy(counts_hbm.at[pl.ds(pl.multiple_of(wid * _L, 8), _L)],
                    cnt_v)
    nblk = cnt_v[pl.ds(0, _L)][0] // _G

    def copy_block(b, idx, dstv, sem):
        off = pl.multiple_of(base + b * _G, 8)
        pltpu.async_copy(csrc_hbm.at[pl.ds(off, _G)], idx, sem)
        pltpu.async_copy(cdst_hbm.at[pl.ds(off, _G)], dstv, sem)

    def wait_block(idx, dstv, sem):
        pltpu.make_async_copy(csrc_hbm.at[pl.ds(0, _G)], idx, sem).wait()
        pltpu.make_async_copy(cdst_hbm.at[pl.ds(0, _G)], dstv, sem).wait()

    def fire_gather(idx, rowsv, sem):
        pltpu.async_copy(x_hbm.at[idx], rowsv, sem)

    def wait_gather(idx, rowsv, sem):
        pltpu.make_async_copy(x_hbm.at[idx], rowsv, sem).wait()

    def scatter_max(dstv, rowsv):
        aggs = (agga, aggb, aggc, aggd)

        def grp_body(jg, c):
            dvec = dstv[pl.ds(pl.multiple_of(jg * _L, _L), _L)]
            for j in range(_L):
                dj = dvec[j]
                rj = jg * _L + j
                dbase = dj * _HD
                for q in range(4):
                    for f in range(_HD // _L):
                        sl = pl.ds(dbase + f * _L, _L)
                        sr = pl.ds(q * _HD + f * _L, _L)
                        aggs[q][sl] = jnp.maximum(aggs[q][sl],
                                                  rowsv[rj, sr])
            return c
        lax.fori_loop(0, _G // _L, grp_body, 0)

    @pl.when(nblk > 0)
    def _():
        off0 = pl.multiple_of(base, 8)
        pltpu.sync_copy(csrc_hbm.at[pl.ds(off0, _G)], idx0)
        pltpu.sync_copy(cdst_hbm.at[pl.ds(off0, _G)], dst0)
        fire_gather(idx0, rows0, semg0)

        @pl.when(nblk > 1)
        def _():
            copy_block(1, idx1, dst1, semi1)

        wait_gather(idx0, rows0, semg0)

        def step(b, cur_idx, cur_dst, cur_rows, cur_semi, cur_semg,
                 nxt_idx, nxt_dst, nxt_rows, nxt_semi, nxt_semg):
            @pl.when(b + 1 < nblk)
            def _():
                wait_block(nxt_idx, nxt_dst, nxt_semi)
                fire_gather(nxt_idx, nxt_rows, nxt_semg)

            scatter_max(cur_dst, cur_rows)

            @pl.when(b + 2 < nblk)
            def _():
                copy_block(b + 2, cur_idx, cur_dst, cur_semi)

            @pl.when(b + 1 < nblk)
            def _():
                wait_gather(nxt_idx, nxt_rows, nxt_semg)

        def blk_body(b, c):
            even = b % 2 == 0

            @pl.when(even)
            def _():
                step(b, idx0, dst0, rows0, semi0, semg0,
                     idx1, dst1, rows1, semi1, semg1)

            @pl.when(jnp.logical_not(even))
            def _():
                step(b, idx1, dst1, rows1, semi1, semg1,
                     idx0, dst0, rows0, semi0, semg0)

            return c

        lax.fori_loop(0, nblk, blk_body, 0)

    aggs4 = (agga, aggb, aggc, aggd)

    def merge_rows(c0, size_rows):
        def mrow(r, c):
            rbase = (c0 + r) * _HD
            for q in range(4):
                for f in range(_HD // _L):
                    v = aggs4[q][pl.ds(rbase + f * _L, _L)]
                    v = jnp.where(v == -jnp.inf, 0.0, v)
                    rows0[r, pl.ds(q * _HD + f * _L, _L)] = v
            return c
        lax.fori_loop(0, size_rows, mrow, 0)

    @pl.when(wid < _NW - 1)
    def _():
        for c0, sz in ((0, 128), (128, 128), (256, 64)):
            merge_rows(c0, sz)
            pltpu.sync_copy(rows0.at[pl.ds(0, sz)],
                            out_hbm.at[pl.ds(lo + c0, sz)])

    @pl.when(wid == _NW - 1)
    def _():
        merge_rows(0, _LAST)
        pltpu.sync_copy(rows0.at[pl.ds(0, _LAST)],
                        out_hbm.at[pl.ds(lo, _LAST)])


def _sc_segmax(x, csrc, cdst, counts):
    f = pl.kernel(
        _sc_segmax_body,
        out_type=jax.ShapeDtypeStruct((_N, _D), jnp.float32),
        mesh=_mesh(),
        scratch_types=[
            pltpu.VMEM((_ROWS_PER_TILE * _HD,), jnp.float32),
            pltpu.VMEM((_ROWS_PER_TILE * _HD,), jnp.float32),
            pltpu.VMEM((_ROWS_PER_TILE * _HD,), jnp.float32),
            pltpu.VMEM((_ROWS_PER_TILE * _HD,), jnp.float32),
            pltpu.VMEM((_G,), jnp.int32),
            pltpu.VMEM((_G,), jnp.int32),
            pltpu.VMEM((_G,), jnp.int32),
            pltpu.VMEM((_G,), jnp.int32),
            pltpu.VMEM((_G, _D), jnp.float32),
            pltpu.VMEM((_G, _D), jnp.float32),
            pltpu.VMEM((_L,), jnp.int32),
            pltpu.SemaphoreType.DMA,
            pltpu.SemaphoreType.DMA,
            pltpu.SemaphoreType.DMA,
            pltpu.SemaphoreType.DMA,
        ],
        compiler_params=_params,
    )
    return f(x, csrc, cdst, counts)


def _tc_linear(x, agg, wt, b, relu):
    def body(x_ref, a_ref, w_ref, b_ref, o_ref):
        acc = jnp.dot(x_ref[...] + a_ref[...], w_ref[...],
                      preferred_element_type=jnp.float32)
        acc = acc + b_ref[...]
        if relu:
            acc = jnp.maximum(acc, 0.0)
        o_ref[...] = acc

    bm = 1000
    return pl.pallas_call(
        body,
        grid=(_N // bm,),
        in_specs=[
            pl.BlockSpec((bm, _D), lambda i: (i, 0)),
            pl.BlockSpec((bm, _D), lambda i: (i, 0)),
            pl.BlockSpec((_D, _D), lambda i: (0, 0)),
            pl.BlockSpec((1, _D), lambda i: (0, 0)),
        ],
        out_specs=pl.BlockSpec((bm, _D), lambda i: (i, 0)),
        out_shape=jax.ShapeDtypeStruct((_N, _D), jnp.float32),
    )(x, agg, wt, b.reshape(1, _D))


def kernel(h, edge_index, W1, b1, W2, b2):
    src = edge_index[0]
    dst = edge_index[1]
    csrc, cdst, counts = _sc_prepass(src, dst)
    agg1 = _sc_segmax(h, csrc, cdst, counts)
    h1 = _tc_linear(h, agg1, W1.T, b1, True)
    agg2 = _sc_segmax(h1, csrc, cdst, counts)
    return _tc_linear(h1, agg2, W2.T, b2, False)


# speedup vs baseline: 1.0329x; 1.0329x over previous
import jax
import jax.numpy as jnp
from jax import lax
from jax.experimental import pallas as pl
from jax.experimental.pallas import tpu as pltpu
from jax.experimental.pallas import tpu_sc as plsc

_N = 10000
_E = 320000
_D = 128

_NC = 2
_NS = 16
_NW = _NC * _NS
_L = 16

_NPW = 320
_LAST = _N - _NPW * (_NW - 1)
_CH = 1600
_U = 10
_G = 256
_RING = 2048
_ROWS_PER_TILE = _NPW + 1
_HD = _D // 4
_CAP = _E + _G

_params = pltpu.CompilerParams(needs_layout_passes=False)


def _mesh():
    return plsc.VectorSubcoreMesh(core_axis_name="c", subcore_axis_name="s",
                                  num_cores=_NC, num_subcores=_NS)


def _sc_prepass_body(src_hbm, dst_hbm, csrc_out, cdst_out, counts_out,
                     csrc, cdst, src_v, dst_v, cnt_v):
    wid = lax.axis_index("s") * _NC + lax.axis_index("c")
    lo = pl.multiple_of(wid * _NPW, 8)
    base = pl.multiple_of(wid * _CAP, 8)

    pad_src = jnp.zeros((_L,), jnp.int32) + lo
    pad_dst = jnp.full((_L,), _NPW, jnp.int32)

    def ring_init(r, c):
        sl = pl.ds(pl.multiple_of(r * _L, _L), _L)
        csrc[sl] = pad_src
        cdst[sl] = pad_dst
        return c
    lax.fori_loop(0, _RING // _L, ring_init, 0)

    lanes = lax.iota(jnp.int32, _L)

    def flush(fired):
        goff = pl.multiple_of(fired & (_RING - 1), _G)
        dst_off = pl.multiple_of(base + fired, 8)
        pltpu.sync_copy(csrc.at[pl.ds(goff, _G)],
                        csrc_out.at[pl.ds(dst_off, _G)])
        pltpu.sync_copy(cdst.at[pl.ds(goff, _G)],
                        cdst_out.at[pl.ds(dst_off, _G)])

    def chunk_body(c, carry):
        pltpu.sync_copy(src_hbm.at[pl.ds(c * _CH, _CH)], src_v)
        pltpu.sync_copy(dst_hbm.at[pl.ds(c * _CH, _CH)], dst_v)

        def scan_body(i, carry2):
            cnt, fired = carry2
            lov = jnp.full((_L,), lo, jnp.int32)
            ms, mis, dls, ss, csums = [], [], [], [], []
            for u in range(_U):
                off = pl.multiple_of((i * _U + u) * _L, _L)
                d = dst_v[pl.ds(off, _L)]
                s = src_v[pl.ds(off, _L)]
                dl = d - lov
                m = (dl >= 0) & (dl < _NPW)
                mi = jnp.where(m, jnp.ones((_L,), jnp.int32),
                               jnp.zeros((_L,), jnp.int32))
                ms.append(m)
                mis.append(mi)
                dls.append(dl)
                ss.append(s)
                csums.append(plsc.cumsum(mi))
            new_cnt = cnt
            for u in range(_U):
                pos = jnp.where(
                    ms[u],
                    (jnp.full((_L,), new_cnt, jnp.int32) + csums[u] - mis[u])
                    & (_RING - 1),
                    jnp.full((_L,), _RING, jnp.int32) + lanes)
                plsc.store_scatter(csrc, [pos], ss[u])
                plsc.store_scatter(cdst, [pos], dls[u])
                new_cnt = new_cnt + csums[u][_L - 1]

            can_flush = new_cnt - fired >= _G

            @pl.when(can_flush)
            def _():
                flush(fired)

            fired = jnp.where(can_flush, fired + _G, fired)
            return new_cnt, fired

        return lax.fori_loop(0, _CH // (_L * _U), scan_body, carry)

    cnt, fired = lax.fori_loop(
        0, _E // _CH, chunk_body, (jnp.int32(0), jnp.int32(0)))

    for _p in range(3):
        do = fired < cnt

        @pl.when(do)
        def _():
            flush(fired)

        fired = jnp.where(do, fired + _G, fired)

    cnt_v[pl.ds(0, _L)] = jnp.full((_L,), fired, jnp.int32)
    pltpu.sync_copy(cnt_v,
                    counts_out.at[pl.ds(pl.multiple_of(wid * _L, 8), _L)])


def _sc_prepass(src, dst):
    f = pl.kernel(
        _sc_prepass_body,
        out_type=(
            jax.ShapeDtypeStruct((_NW * _CAP,), jnp.int32),
            jax.ShapeDtypeStruct((_NW * _CAP,), jnp.int32),
            jax.ShapeDtypeStruct((_NW * _L,), jnp.int32),
        ),
        mesh=_mesh(),
        scratch_types=[
            pltpu.VMEM((_RING + _L,), jnp.int32),
            pltpu.VMEM((_RING + _L,), jnp.int32),
            pltpu.VMEM((_CH,), jnp.int32),
            pltpu.VMEM((_CH,), jnp.int32),
            pltpu.VMEM((_L,), jnp.int32),
        ],
        compiler_params=_params,
    )
    return f(src, dst)


def _sc_segmax_body(x_hbm, csrc_hbm, cdst_hbm, counts_hbm, out_hbm,
                    agga, aggb, aggc, aggd,
                    idx0, idx1, dst0, dst1, rows0, rows1, cnt_v,
                    semi0, semi1, semg0, semg1):
    wid = lax.axis_index("s") * _NC + lax.axis_index("c")
    lo = pl.multiple_of(wid * _NPW, 8)
    base = pl.multiple_of(wid * _CAP, 8)

    neg_inf = jnp.full((_L,), -jnp.inf, jnp.float32)

    def init_body(i, c):
        sl = pl.ds(pl.multiple_of(i * _L, _L), _L)
        agga[sl] = neg_inf
        aggb[sl] = neg_inf
        aggc[sl] = neg_inf
        aggd[sl] = neg_inf
        return c
    lax.fori_loop(0, _ROWS_PER_TILE * _HD // _L, init_body, 0)

    pltpu.sync_copy(counts_hbm.at[pl.ds(pl.multiple_of(wid * _L, 8), _L)],
                    cnt_v)
    nblk = cnt_v[pl.ds(0, _L)][0] // _G

    def copy_block(b, idx, dstv, sem):
        off = pl.multiple_of(base + b * _G, 8)
        pltpu.async_copy(csrc_hbm.at[pl.ds(off, _G)], idx, sem)
        pltpu.async_copy(cdst_hbm.at[pl.ds(off, _G)], dstv, sem)

    def wait_block(idx, dstv, sem):
        pltpu.make_async_copy(csrc_hbm.at[pl.ds(0, _G)], idx, sem).wait()
        pltpu.make_async_copy(cdst_hbm.at[pl.ds(0, _G)], dstv, sem).wait()

    def fire_gather(idx, rowsv, sem):
        pltpu.async_copy(x_hbm.at[idx], rowsv, sem)

    def wait_gather(idx, rowsv, sem):
        pltpu.make_async_copy(x_hbm.at[idx], rowsv, sem).wait()

    def scatter_max(dstv, rowsv):
        aggs = (agga, aggb, aggc, aggd)

        def grp_body(jg, c):
            dvec = dstv[pl.ds(pl.multiple_of(jg * _L, _L), _L)]
            for j in range(_L):
                dj = dvec[j]
                rj = jg * _L + j
                dbase = dj * _HD
                for q in range(4):
                    for f in range(_HD // _L):
                        sl = pl.ds(dbase + f * _L, _L)
                        sr = pl.ds(q * _HD + f * _L, _L)
                        aggs[q][sl] = jnp.maximum(aggs[q][sl],
                                                  rowsv[rj, sr])
            return c
        lax.fori_loop(0, _G // _L, grp_body, 0)

    @pl.when(nblk > 0)
    def _():
        off0 = pl.multiple_of(base, 8)
        pltpu.sync_copy(csrc_hbm.at[pl.ds(off0, _G)], idx0)
        pltpu.sync_copy(cdst_hbm.at[pl.ds(off0, _G)], dst0)
        fire_gather(idx0, rows0, semg0)

        @pl.when(nblk > 1)
        def _():
            copy_block(1, idx1, dst1, semi1)

        wait_gather(idx0, rows0, semg0)

        def step(b, cur_idx, cur_dst, cur_rows, cur_semi, cur_semg,
                 nxt_idx, nxt_dst, nxt_rows, nxt_semi, nxt_semg):
            @pl.when(b + 1 < nblk)
            def _():
                wait_block(nxt_idx, nxt_dst, nxt_semi)
                fire_gather(nxt_idx, nxt_rows, nxt_semg)

            scatter_max(cur_dst, cur_rows)

            @pl.when(b + 2 < nblk)
            def _():
                copy_block(b + 2, cur_idx, cur_dst, cur_semi)

            @pl.when(b + 1 < nblk)
            def _():
                wait_gather(nxt_idx, nxt_rows, nxt_semg)

        def blk_body(b, c):
            even = b % 2 == 0

            @pl.when(even)
            def _():
                step(b, idx0, dst0, rows0, semi0, semg0,
                     idx1, dst1, rows1, semi1, semg1)

            @pl.when(jnp.logical_not(even))
            def _():
                step(b, idx1, dst1, rows1, semi1, semg1,
                     idx0, dst0, rows0, semi0, semg0)

            return c

        lax.fori_loop(0, nblk, blk_body, 0)

    aggs4 = (agga, aggb, aggc, aggd)

    def merge_rows(c0, size_rows):
        def mrow(r, c):
            rbase = (c0 + r) * _HD
            for q in range(4):
                for f in range(_HD // _L):
                    v = aggs4[q][pl.ds(rbase + f * _L, _L)]
                    v = jnp.where(v == -jnp.inf, 0.0, v)
                    rows0[r, pl.ds(q * _HD + f * _L, _L)] = v
            return c
        lax.fori_loop(0, size_rows, mrow, 0)

    @pl.when(wid < _NW - 1)
    def _():
        for c0, sz in ((0, 128), (128, 128), (256, 64)):
            merge_rows(c0, sz)
            pltpu.sync_copy(rows0.at[pl.ds(0, sz)],
                            out_hbm.at[pl.ds(lo + c0, sz)])

    @pl.when(wid == _NW - 1)
    def _():
        merge_rows(0, _LAST)
        pltpu.sync_copy(rows0.at[pl.ds(0, _LAST)],
                        out_hbm.at[pl.ds(lo, _LAST)])


def _sc_segmax(x, csrc, cdst, counts):
    f = pl.kernel(
        _sc_segmax_body,
        out_type=jax.ShapeDtypeStruct((_N, _D), jnp.float32),
        mesh=_mesh(),
        scratch_types=[
            pltpu.VMEM((_ROWS_PER_TILE * _HD,), jnp.float32),
            pltpu.VMEM((_ROWS_PER_TILE * _HD,), jnp.float32),
            pltpu.VMEM((_ROWS_PER_TILE * _HD,), jnp.float32),
            pltpu.VMEM((_ROWS_PER_TILE * _HD,), jnp.float32),
            pltpu.VMEM((_G,), jnp.int32),
            pltpu.VMEM((_G,), jnp.int32),
            pltpu.VMEM((_G,), jnp.int32),
            pltpu.VMEM((_G,), jnp.int32),
            pltpu.VMEM((_G, _D), jnp.float32),
            pltpu.VMEM((_G, _D), jnp.float32),
            pltpu.VMEM((_L,), jnp.int32),
            pltpu.SemaphoreType.DMA,
            pltpu.SemaphoreType.DMA,
            pltpu.SemaphoreType.DMA,
            pltpu.SemaphoreType.DMA,
        ],
        compiler_params=_params,
    )
    return f(x, csrc, cdst, counts)


def _tc_linear(x, agg, wt, b, relu):
    def body(x_ref, a_ref, w_ref, b_ref, o_ref):
        acc = jnp.dot(x_ref[...] + a_ref[...], w_ref[...],
                      preferred_element_type=jnp.float32)
        acc = acc + b_ref[...]
        if relu:
            acc = jnp.maximum(acc, 0.0)
        o_ref[...] = acc

    bm = 1000
    return pl.pallas_call(
        body,
        grid=(_N // bm,),
        in_specs=[
            pl.BlockSpec((bm, _D), lambda i: (i, 0)),
            pl.BlockSpec((bm, _D), lambda i: (i, 0)),
            pl.BlockSpec((_D, _D), lambda i: (0, 0)),
            pl.BlockSpec((1, _D), lambda i: (0, 0)),
        ],
        out_specs=pl.BlockSpec((bm, _D), lambda i: (i, 0)),
        out_shape=jax.ShapeDtypeStruct((_N, _D), jnp.float32),
    )(x, agg, wt, b.reshape(1, _D))


def kernel(h, edge_index, W1, b1, W2, b2):
    src = edge_index[0]
    dst = edge_index[1]
    csrc, cdst, counts = _sc_prepass(src, dst)
    agg1 = _sc_segmax(h, csrc, cdst, counts)
    h1 = _tc_linear(h, agg1, W1.T, b1, True)
    agg2 = _sc_segmax(h1, csrc, cdst, counts)
    return _tc_linear(h1, agg2, W2.T, b2, False)


# speedup vs baseline: 1.0425x; 1.0093x over previous
import jax
import jax.numpy as jnp
from jax import lax
from jax.experimental import pallas as pl
from jax.experimental.pallas import tpu as pltpu
from jax.experimental.pallas import tpu_sc as plsc

_N = 10000
_E = 320000
_D = 128

_NC = 2
_NS = 16
_NW = _NC * _NS
_L = 16

_NPW = 320
_LAST = _N - _NPW * (_NW - 1)
_CH = 1600
_U = 10
_G = 256
_RING = 2048
_ROWS_PER_TILE = _NPW + 1
_HD = _D // 4
_CAP = _E + _G

_params = pltpu.CompilerParams(needs_layout_passes=False)


def _mesh():
    return plsc.VectorSubcoreMesh(core_axis_name="c", subcore_axis_name="s",
                                  num_cores=_NC, num_subcores=_NS)


def _sc_prepass_body(x_hbm, src_hbm, dst_hbm,
                     csrc_out, cdst_out, counts_out, out_hbm,
                     csrc, cdst, src_v, dst_v, cnt_v,
                     agga, aggb, aggc, aggd, rows0, rows1,
                     semg0, semg1):
    wid = lax.axis_index("s") * _NC + lax.axis_index("c")
    lo = pl.multiple_of(wid * _NPW, 8)
    base = pl.multiple_of(wid * _CAP, 8)

    neg_inf = jnp.full((_L,), -jnp.inf, jnp.float32)
    aggs = (agga, aggb, aggc, aggd)

    def init_body(i, c):
        sl = pl.ds(pl.multiple_of(i * _L, _L), _L)
        agga[sl] = neg_inf
        aggb[sl] = neg_inf
        aggc[sl] = neg_inf
        aggd[sl] = neg_inf
        return c
    lax.fori_loop(0, _ROWS_PER_TILE * _HD // _L, init_body, 0)

    pad_src = jnp.zeros((_L,), jnp.int32) + lo
    pad_dst = jnp.full((_L,), _NPW, jnp.int32)

    def ring_init(r, c):
        sl = pl.ds(pl.multiple_of(r * _L, _L), _L)
        csrc[sl] = pad_src
        cdst[sl] = pad_dst
        return c
    lax.fori_loop(0, _RING // _L, ring_init, 0)

    lanes = lax.iota(jnp.int32, _L)

    def scatter_max(goff, rowsv):
        def grp_body(jg, c):
            dvec = cdst[pl.ds(pl.multiple_of(goff + jg * _L, _L), _L)]
            for j in range(_L):
                dj = dvec[j]
                rj = jg * _L + j
                dbase = dj * _HD
                for q in range(4):
                    for f in range(_HD // _L):
                        sl = pl.ds(dbase + f * _L, _L)
                        sr = pl.ds(q * _HD + f * _L, _L)
                        aggs[q][sl] = jnp.maximum(aggs[q][sl],
                                                  rowsv[rj, sr])
            return c
        lax.fori_loop(0, _G // _L, grp_body, 0)

    def finish_prev(fired):
        pgoff = pl.multiple_of((fired - _G) & (_RING - 1), _G)
        podd = ((fired - _G) & _G) != 0

        @pl.when(jnp.logical_not(podd))
        def _():
            pltpu.make_async_copy(
                x_hbm.at[csrc.at[pl.ds(pgoff, _G)]], rows0, semg0).wait()
            scatter_max(pgoff, rows0)

        @pl.when(podd)
        def _():
            pltpu.make_async_copy(
                x_hbm.at[csrc.at[pl.ds(pgoff, _G)]], rows1, semg1).wait()
            scatter_max(pgoff, rows1)

    def fire(fired):
        goff = pl.multiple_of(fired & (_RING - 1), _G)
        dst_off = pl.multiple_of(base + fired, 8)

        @pl.when(fired > 0)
        def _():
            finish_prev(fired)

        pltpu.sync_copy(csrc.at[pl.ds(goff, _G)],
                        csrc_out.at[pl.ds(dst_off, _G)])
        pltpu.sync_copy(cdst.at[pl.ds(goff, _G)],
                        cdst_out.at[pl.ds(dst_off, _G)])

        odd = (fired & _G) != 0

        @pl.when(jnp.logical_not(odd))
        def _():
            pltpu.async_copy(
                x_hbm.at[csrc.at[pl.ds(goff, _G)]], rows0, semg0)

        @pl.when(odd)
        def _():
            pltpu.async_copy(
                x_hbm.at[csrc.at[pl.ds(goff, _G)]], rows1, semg1)

    def chunk_body(c, carry):
        pltpu.sync_copy(src_hbm.at[pl.ds(c * _CH, _CH)], src_v)
        pltpu.sync_copy(dst_hbm.at[pl.ds(c * _CH, _CH)], dst_v)

        def scan_body(i, carry2):
            cnt, fired = carry2
            lov = jnp.full((_L,), lo, jnp.int32)
            ms, mis, dls, ss, csums = [], [], [], [], []
            for u in range(_U):
                off = pl.multiple_of((i * _U + u) * _L, _L)
                d = dst_v[pl.ds(off, _L)]
                s2 = src_v[pl.ds(off, _L)]
                dl = d - lov
                m = (dl >= 0) & (dl < _NPW)
                mi = jnp.where(m, jnp.ones((_L,), jnp.int32),
                               jnp.zeros((_L,), jnp.int32))
                ms.append(m)
                mis.append(mi)
                dls.append(dl)
                ss.append(s2)
                csums.append(plsc.cumsum(mi))
            new_cnt = cnt
            for u in range(_U):
                pos = jnp.where(
                    ms[u],
                    (jnp.full((_L,), new_cnt, jnp.int32) + csums[u] - mis[u])
                    & (_RING - 1),
                    jnp.full((_L,), _RING, jnp.int32) + lanes)
                plsc.store_scatter(csrc, [pos], ss[u])
                plsc.store_scatter(cdst, [pos], dls[u])
                new_cnt = new_cnt + csums[u][_L - 1]

            can_flush = new_cnt - fired >= _G

            @pl.when(can_flush)
            def _():
                fire(fired)

            fired = jnp.where(can_flush, fired + _G, fired)
            return new_cnt, fired

        return lax.fori_loop(0, _CH // (_L * _U), scan_body, carry)

    cnt, fired = lax.fori_loop(
        0, _E // _CH, chunk_body, (jnp.int32(0), jnp.int32(0)))

    for _p in range(3):
        do = fired < cnt

        @pl.when(do)
        def _():
            fire(fired)

        fired = jnp.where(do, fired + _G, fired)

    @pl.when(fired > 0)
    def _():
        finish_prev(fired)

    cnt_v[pl.ds(0, _L)] = jnp.full((_L,), fired, jnp.int32)
    pltpu.sync_copy(cnt_v,
                    counts_out.at[pl.ds(pl.multiple_of(wid * _L, 8), _L)])

    def merge_rows(c0, size_rows):
        def mrow(r, c):
            rbase = (c0 + r) * _HD
            for q in range(4):
                for f in range(_HD // _L):
                    v = aggs[q][pl.ds(rbase + f * _L, _L)]
                    v = jnp.where(v == -jnp.inf, 0.0, v)
                    rows0[r, pl.ds(q * _HD + f * _L, _L)] = v
            return c
        lax.fori_loop(0, size_rows, mrow, 0)

    @pl.when(wid < _NW - 1)
    def _():
        for c0, sz in ((0, 128), (128, 128), (256, 64)):
            merge_rows(c0, sz)
            pltpu.sync_copy(rows0.at[pl.ds(0, sz)],
                            out_hbm.at[pl.ds(lo + c0, sz)])

    @pl.when(wid == _NW - 1)
    def _():
        merge_rows(0, _LAST)
        pltpu.sync_copy(rows0.at[pl.ds(0, _LAST)],
                        out_hbm.at[pl.ds(lo, _LAST)])


def _sc_prepass(x, src, dst):
    f = pl.kernel(
        _sc_prepass_body,
        out_type=(
            jax.ShapeDtypeStruct((_NW * _CAP,), jnp.int32),
            jax.ShapeDtypeStruct((_NW * _CAP,), jnp.int32),
            jax.ShapeDtypeStruct((_NW * _L,), jnp.int32),
            jax.ShapeDtypeStruct((_N, _D), jnp.float32),
        ),
        mesh=_mesh(),
        scratch_types=[
            pltpu.VMEM((_RING + _L,), jnp.int32),
            pltpu.VMEM((_RING + _L,), jnp.int32),
            pltpu.VMEM((_CH,), jnp.int32),
            pltpu.VMEM((_CH,), jnp.int32),
            pltpu.VMEM((_L,), jnp.int32),
            pltpu.VMEM((_ROWS_PER_TILE * _HD,), jnp.float32),
            pltpu.VMEM((_ROWS_PER_TILE * _HD,), jnp.float32),
            pltpu.VMEM((_ROWS_PER_TILE * _HD,), jnp.float32),
            pltpu.VMEM((_ROWS_PER_TILE * _HD,), jnp.float32),
            pltpu.VMEM((_G, _D), jnp.float32),
            pltpu.VMEM((_G, _D), jnp.float32),
            pltpu.SemaphoreType.DMA,
            pltpu.SemaphoreType.DMA,
        ],
        compiler_params=_params,
    )
    return f(x, src, dst)


def _sc_segmax_body(x_hbm, csrc_hbm, cdst_hbm, counts_hbm, out_hbm,
                    agga, aggb, aggc, aggd,
                    idx0, idx1, dst0, dst1, rows0, rows1, cnt_v,
                    semi0, semi1, semg0, semg1):
    wid = lax.axis_index("s") * _NC + lax.axis_index("c")
    lo = pl.multiple_of(wid * _NPW, 8)
    base = pl.multiple_of(wid * _CAP, 8)

    neg_inf = jnp.full((_L,), -jnp.inf, jnp.float32)

    def init_body(i, c):
        sl = pl.ds(pl.multiple_of(i * _L, _L), _L)
        agga[sl] = neg_inf
        aggb[sl] = neg_inf
        aggc[sl] = neg_inf
        aggd[sl] = neg_inf
        return c
    lax.fori_loop(0, _ROWS_PER_TILE * _HD // _L, init_body, 0)

    pltpu.sync_copy(counts_hbm.at[pl.ds(pl.multiple_of(wid * _L, 8), _L)],
                    cnt_v)
    nblk = cnt_v[pl.ds(0, _L)][0] // _G

    def copy_block(b, idx, dstv, sem):
        off = pl.multiple_of(base + b * _G, 8)
        pltpu.async_copy(csrc_hbm.at[pl.ds(off, _G)], idx, sem)
        pltpu.async_copy(cdst_hbm.at[pl.ds(off, _G)], dstv, sem)

    def wait_block(idx, dstv, sem):
        pltpu.make_async_copy(csrc_hbm.at[pl.ds(0, _G)], idx, sem).wait()
        pltpu.make_async_copy(cdst_hbm.at[pl.ds(0, _G)], dstv, sem).wait()

    def fire_gather(idx, rowsv, sem):
        pltpu.async_copy(x_hbm.at[idx], rowsv, sem)

    def wait_gather(idx, rowsv, sem):
        pltpu.make_async_copy(x_hbm.at[idx], rowsv, sem).wait()

    def scatter_max(dstv, rowsv):
        aggs = (agga, aggb, aggc, aggd)

        def grp_body(jg, c):
            dvec = dstv[pl.ds(pl.multiple_of(jg * _L, _L), _L)]
            for j in range(_L):
                dj = dvec[j]
                rj = jg * _L + j
                dbase = dj * _HD
                for q in range(4):
                    for f in range(_HD // _L):
                        sl = pl.ds(dbase + f * _L, _L)
                        sr = pl.ds(q * _HD + f * _L, _L)
                        aggs[q][sl] = jnp.maximum(aggs[q][sl],
                                                  rowsv[rj, sr])
            return c
        lax.fori_loop(0, _G // _L, grp_body, 0)

    @pl.when(nblk > 0)
    def _():
        off0 = pl.multiple_of(base, 8)
        pltpu.sync_copy(csrc_hbm.at[pl.ds(off0, _G)], idx0)
        pltpu.sync_copy(cdst_hbm.at[pl.ds(off0, _G)], dst0)
        fire_gather(idx0, rows0, semg0)

        @pl.when(nblk > 1)
        def _():
            copy_block(1, idx1, dst1, semi1)

        wait_gather(idx0, rows0, semg0)

        def step(b, cur_idx, cur_dst, cur_rows, cur_semi, cur_semg,
                 nxt_idx, nxt_dst, nxt_rows, nxt_semi, nxt_semg):
            @pl.when(b + 1 < nblk)
            def _():
                wait_block(nxt_idx, nxt_dst, nxt_semi)
                fire_gather(nxt_idx, nxt_rows, nxt_semg)

            scatter_max(cur_dst, cur_rows)

            @pl.when(b + 2 < nblk)
            def _():
                copy_block(b + 2, cur_idx, cur_dst, cur_semi)

            @pl.when(b + 1 < nblk)
            def _():
                wait_gather(nxt_idx, nxt_rows, nxt_semg)

        def blk_body(b, c):
            even = b % 2 == 0

            @pl.when(even)
            def _():
                step(b, idx0, dst0, rows0, semi0, semg0,
                     idx1, dst1, rows1, semi1, semg1)

            @pl.when(jnp.logical_not(even))
            def _():
                step(b, idx1, dst1, rows1, semi1, semg1,
                     idx0, dst0, rows0, semi0, semg0)

            return c

        lax.fori_loop(0, nblk, blk_body, 0)

    aggs4 = (agga, aggb, aggc, aggd)

    def merge_rows(c0, size_rows):
        def mrow(r, c):
            rbase = (c0 + r) * _HD
            for q in range(4):
                for f in range(_HD // _L):
                    v = aggs4[q][pl.ds(rbase + f * _L, _L)]
                    v = jnp.where(v == -jnp.inf, 0.0, v)
                    rows0[r, pl.ds(q * _HD + f * _L, _L)] = v
            return c
        lax.fori_loop(0, size_rows, mrow, 0)

    @pl.when(wid < _NW - 1)
    def _():
        for c0, sz in ((0, 128), (128, 128), (256, 64)):
            merge_rows(c0, sz)
            pltpu.sync_copy(rows0.at[pl.ds(0, sz)],
                            out_hbm.at[pl.ds(lo + c0, sz)])

    @pl.when(wid == _NW - 1)
    def _():
        merge_rows(0, _LAST)
        pltpu.sync_copy(rows0.at[pl.ds(0, _LAST)],
                        out_hbm.at[pl.ds(lo, _LAST)])


def _sc_segmax(x, csrc, cdst, counts):
    f = pl.kernel(
        _sc_segmax_body,
        out_type=jax.ShapeDtypeStruct((_N, _D), jnp.float32),
        mesh=_mesh(),
        scratch_types=[
            pltpu.VMEM((_ROWS_PER_TILE * _HD,), jnp.float32),
            pltpu.VMEM((_ROWS_PER_TILE * _HD,), jnp.float32),
            pltpu.VMEM((_ROWS_PER_TILE * _HD,), jnp.float32),
            pltpu.VMEM((_ROWS_PER_TILE * _HD,), jnp.float32),
            pltpu.VMEM((_G,), jnp.int32),
            pltpu.VMEM((_G,), jnp.int32),
            pltpu.VMEM((_G,), jnp.int32),
            pltpu.VMEM((_G,), jnp.int32),
            pltpu.VMEM((_G, _D), jnp.float32),
            pltpu.VMEM((_G, _D), jnp.float32),
            pltpu.VMEM((_L,), jnp.int32),
            pltpu.SemaphoreType.DMA,
            pltpu.SemaphoreType.DMA,
            pltpu.SemaphoreType.DMA,
            pltpu.SemaphoreType.DMA,
        ],
        compiler_params=_params,
    )
    return f(x, csrc, cdst, counts)


def _tc_linear(x, agg, wt, b, relu):
    def body(x_ref, a_ref, w_ref, b_ref, o_ref):
        acc = jnp.dot(x_ref[...] + a_ref[...], w_ref[...],
                      preferred_element_type=jnp.float32)
        acc = acc + b_ref[...]
        if relu:
            acc = jnp.maximum(acc, 0.0)
        o_ref[...] = acc

    bm = 1000
    return pl.pallas_call(
        body,
        grid=(_N // bm,),
        in_specs=[
            pl.BlockSpec((bm, _D), lambda i: (i, 0)),
            pl.BlockSpec((bm, _D), lambda i: (i, 0)),
            pl.BlockSpec((_D, _D), lambda i: (0, 0)),
            pl.BlockSpec((1, _D), lambda i: (0, 0)),
        ],
        out_specs=pl.BlockSpec((bm, _D), lambda i: (i, 0)),
        out_shape=jax.ShapeDtypeStruct((_N, _D), jnp.float32),
    )(x, agg, wt, b.reshape(1, _D))


def kernel(h, edge_index, W1, b1, W2, b2):
    src = edge_index[0]
    dst = edge_index[1]
    csrc, cdst, counts, agg1 = _sc_prepass(h, src, dst)
    h1 = _tc_linear(h, agg1, W1.T, b1, True)
    agg2 = _sc_segmax(h1, csrc, cdst, counts)
    return _tc_linear(h1, agg2, W2.T, b2, False)
